# initial kernel scaffold (unmeasured)
import jax
import jax.numpy as jnp
from jax import lax
from jax.experimental import pallas as pl
from jax.experimental.pallas import tpu as pltpu

N_DEV = 4
RING_COLS = 2048
TILE_COLS = 1024
N_HOPS = N_DEV - 1


def kernel(x, w_mat):
    m_g, k_l = x.shape
    k_l2, n = w_mat.shape
    assert k_l == k_l2
    m_l = m_g // N_DEV
    n_rings = n // RING_COLS
    nt = RING_COLS // TILE_COLS

    x = x.astype(jnp.bfloat16)
    w = w_mat.astype(jnp.bfloat16)

    def body(x_ref, w_ref, out_ref, slots, send_sems, recv_sems,
             ax_slots, ax_send_sems, ax_recv_sems, copy_sem):
        my = lax.axis_index("i")
        left = lax.rem(my + N_DEV - 1, N_DEV)
        right = lax.rem(my + 1, N_DEV)

        barrier = pltpu.get_barrier_semaphore()

        def neighbor_barrier():
            for nbr in (left, right):
                pl.semaphore_signal(
                    barrier, inc=1, device_id=(nbr,),
                    device_id_type=pl.DeviceIdType.MESH,
                )
            pl.semaphore_wait(barrier, 2)

        def partial_tiles(b, c0):
            xs = x_ref[pl.ds(b * m_l, m_l), :]
            for t in range(nt):
                wt = w_ref[:, c0 + t * TILE_COLS: c0 + (t + 1) * TILE_COLS]
                yield t, jnp.dot(xs, wt, preferred_element_type=jnp.float32)

        def tcols(t):
            return slice(t * TILE_COLS, (t + 1) * TILE_COLS)

        neighbor_barrier()

        amax = jnp.float32(0.0)

        for r in range(n_rings):
            c0 = r * RING_COLS
            if r > 0:
                neighbor_barrier()

            b0 = lax.rem(my + N_DEV - 1, N_DEV)
            for t, p in partial_tiles(b0, c0):
                slots[0, :, tcols(t)] = p.astype(jnp.bfloat16)

            for h in range(N_HOPS):
                rdma = pltpu.make_async_remote_copy(
                    src_ref=slots.at[h],
                    dst_ref=slots.at[h + 1],
                    send_sem=send_sems.at[h],
                    recv_sem=recv_sems.at[h],
                    device_id=(right,),
                    device_id_type=pl.DeviceIdType.MESH,
                )
                rdma.start()
                rdma.wait()
                if h < N_HOPS - 1:
                    b = lax.rem(my + 2 * N_DEV - 2 - h, N_DEV)
                    for t, p in partial_tiles(b, c0):
                        acc = slots[h + 1, :, tcols(t)].astype(jnp.float32) + p
                        slots[h + 1, :, tcols(t)] = acc.astype(jnp.bfloat16)

            for t, p in partial_tiles(my, c0):
                y32 = slots[N_HOPS, :, tcols(t)].astype(jnp.float32) + p
                amax = jnp.maximum(amax, jnp.max(jnp.abs(y32)))
                slots[N_HOPS, :, tcols(t)] = y32.astype(jnp.bfloat16)

            cp = pltpu.make_async_copy(
                slots.at[N_HOPS], out_ref.at[:, pl.ds(c0, RING_COLS)], copy_sem
            )
            cp.start()
            cp.wait()

        own = jnp.full((8, 128), amax, jnp.float32)
        ax_slots[0, :, :] = own
        for h in range(N_HOPS):
            rdma = pltpu.make_async_remote_copy(
                src_ref=ax_slots.at[h],
                dst_ref=ax_slots.at[h + 1],
                send_sem=ax_send_sems.at[h],
                recv_sem=ax_recv_sems.at[h],
                device_id=(right,),
                device_id_type=pl.DeviceIdType.MESH,
            )
            rdma.start()
            rdma.wait()
            ax_slots[h + 1, :, :] = jnp.maximum(ax_slots[h + 1, :, :], own)
        g = jnp.max(ax_slots[N_HOPS, :, :])

        scale = g / 448.0
        inv = 448.0 / g
        for r in range(n_rings):
            c0 = r * RING_COLS
            cp = pltpu.make_async_copy(
                out_ref.at[:, pl.ds(c0, RING_COLS)], slots.at[0], copy_sem
            )
            cp.start()
            cp.wait()
            for t in range(nt):
                v = slots[0, :, tcols(t)].astype(jnp.float32) * inv
                v = jnp.clip(v, -448.0, 448.0)
                q = v.astype(jnp.float8_e4m3fn).astype(jnp.float32)
                slots[0, :, tcols(t)] = (q * scale).astype(jnp.bfloat16)
            cp = pltpu.make_async_copy(
                slots.at[0], out_ref.at[:, pl.ds(c0, RING_COLS)], copy_sem
            )
            cp.start()
            cp.wait()

    return pl.pallas_call(
        body,
        out_shape=jax.ShapeDtypeStruct((m_l, n), jnp.bfloat16),
        in_specs=[
            pl.BlockSpec(memory_space=pltpu.VMEM),
            pl.BlockSpec(memory_space=pltpu.VMEM),
        ],
        out_specs=pl.BlockSpec(memory_space=pltpu.ANY),
        scratch_shapes=[
            pltpu.VMEM((N_HOPS + 1, m_l, RING_COLS), jnp.bfloat16),
            pltpu.SemaphoreType.DMA((N_HOPS,)),
            pltpu.SemaphoreType.DMA((N_HOPS,)),
            pltpu.VMEM((N_HOPS + 1, 8, 128), jnp.float32),
            pltpu.SemaphoreType.DMA((N_HOPS,)),
            pltpu.SemaphoreType.DMA((N_HOPS,)),
            pltpu.SemaphoreType.DMA,
        ],
        compiler_params=pltpu.CompilerParams(collective_id=0),
    )(x, w)


# baseline (device time: 728269 ns/iter reference)
import jax
import jax.numpy as jnp
from jax import lax
from jax.experimental import pallas as pl
from jax.experimental.pallas import tpu as pltpu

N_DEV = 4
RING_COLS = 2048
TILE_COLS = 1024
N_HOPS = N_DEV - 1


def kernel(x, w_mat):
    m_g, k_l = x.shape
    k_l2, n = w_mat.shape
    assert k_l == k_l2
    m_l = m_g // N_DEV
    n_rings = n // RING_COLS
    nt = RING_COLS // TILE_COLS

    x = x.astype(jnp.bfloat16)
    w = w_mat.astype(jnp.bfloat16)

    def body(x_ref, w_ref, out_ref, slots, send_sems, recv_sems,
             ax_slots, ax_send_sems, ax_recv_sems, copy_sem):
        my = lax.axis_index("i")
        left = lax.rem(my + N_DEV - 1, N_DEV)
        right = lax.rem(my + 1, N_DEV)

        barrier = pltpu.get_barrier_semaphore()

        def neighbor_barrier():
            for nbr in (left, right):
                pl.semaphore_signal(
                    barrier, inc=1, device_id=(nbr,),
                    device_id_type=pl.DeviceIdType.MESH,
                )
            pl.semaphore_wait(barrier, 2)

        def partial_tiles(b, c0):
            xs = x_ref[pl.ds(b * m_l, m_l), :]
            for t in range(nt):
                wt = w_ref[:, c0 + t * TILE_COLS: c0 + (t + 1) * TILE_COLS]
                yield t, jnp.dot(xs, wt, preferred_element_type=jnp.float32)

        def tcols(t):
            return slice(t * TILE_COLS, (t + 1) * TILE_COLS)

        neighbor_barrier()

        amax = jnp.float32(0.0)

        for r in range(n_rings):
            c0 = r * RING_COLS
            if r > 0:
                neighbor_barrier()

            b0 = lax.rem(my + N_DEV - 1, N_DEV)
            for t, p in partial_tiles(b0, c0):
                slots[0, :, tcols(t)] = p.astype(jnp.bfloat16)

            for h in range(N_HOPS):
                rdma = pltpu.make_async_remote_copy(
                    src_ref=slots.at[h],
                    dst_ref=slots.at[h + 1],
                    send_sem=send_sems.at[h],
                    recv_sem=recv_sems.at[h],
                    device_id=(right,),
                    device_id_type=pl.DeviceIdType.MESH,
                )
                rdma.start()
                rdma.wait()
                if h < N_HOPS - 1:
                    b = lax.rem(my + 2 * N_DEV - 2 - h, N_DEV)
                    for t, p in partial_tiles(b, c0):
                        acc = slots[h + 1, :, tcols(t)].astype(jnp.float32) + p
                        slots[h + 1, :, tcols(t)] = acc.astype(jnp.bfloat16)

            for t, p in partial_tiles(my, c0):
                y32 = slots[N_HOPS, :, tcols(t)].astype(jnp.float32) + p
                amax = jnp.maximum(amax, jnp.max(jnp.abs(y32)))
                slots[N_HOPS, :, tcols(t)] = y32.astype(jnp.bfloat16)

            cp = pltpu.make_async_copy(
                slots.at[N_HOPS], out_ref.at[:, pl.ds(c0, RING_COLS)], copy_sem
            )
            cp.start()
            cp.wait()

        own = jnp.full((8, 128), amax, jnp.float32)
        ax_slots[0, :, :] = own
        for h in range(N_HOPS):
            rdma = pltpu.make_async_remote_copy(
                src_ref=ax_slots.at[h],
                dst_ref=ax_slots.at[h + 1],
                send_sem=ax_send_sems.at[h],
                recv_sem=ax_recv_sems.at[h],
                device_id=(right,),
                device_id_type=pl.DeviceIdType.MESH,
            )
            rdma.start()
            rdma.wait()
            ax_slots[h + 1, :, :] = jnp.maximum(ax_slots[h + 1, :, :], own)
        g = jnp.max(ax_slots[N_HOPS, :, :])

        scale = g / 448.0
        inv = 448.0 / g
        for r in range(n_rings):
            c0 = r * RING_COLS
            cp = pltpu.make_async_copy(
                out_ref.at[:, pl.ds(c0, RING_COLS)], slots.at[0], copy_sem
            )
            cp.start()
            cp.wait()
            for t in range(nt):
                v = slots[0, :, tcols(t)].astype(jnp.float32) * inv
                v = jnp.clip(v, -448.0, 448.0)
                q = v.astype(jnp.float8_e4m3fn).astype(jnp.float32)
                slots[0, :, tcols(t)] = (q * scale).astype(jnp.bfloat16)
            cp = pltpu.make_async_copy(
                slots.at[0], out_ref.at[:, pl.ds(c0, RING_COLS)], copy_sem
            )
            cp.start()
            cp.wait()

    return pl.pallas_call(
        body,
        out_shape=jax.ShapeDtypeStruct((m_l, n), jnp.bfloat16),
        in_specs=[
            pl.BlockSpec(memory_space=pltpu.VMEM),
            pl.BlockSpec(memory_space=pltpu.VMEM),
        ],
        out_specs=pl.BlockSpec(memory_space=pl.ANY),
        scratch_shapes=[
            pltpu.VMEM((N_HOPS + 1, m_l, RING_COLS), jnp.bfloat16),
            pltpu.SemaphoreType.DMA((N_HOPS,)),
            pltpu.SemaphoreType.DMA((N_HOPS,)),
            pltpu.VMEM((N_HOPS + 1, 8, 128), jnp.float32),
            pltpu.SemaphoreType.DMA((N_HOPS,)),
            pltpu.SemaphoreType.DMA((N_HOPS,)),
            pltpu.SemaphoreType.DMA,
        ],
        compiler_params=pltpu.CompilerParams(
            collective_id=0, vmem_limit_bytes=64 * 1024 * 1024
        ),
    )(x, w)


# device time: 466637 ns/iter; 1.5607x vs baseline; 1.5607x over previous
import jax
import jax.numpy as jnp
from jax import lax
from jax.experimental import pallas as pl
from jax.experimental.pallas import tpu as pltpu

N_DEV = 4
RING_COLS = 1024
N_HOPS = N_DEV - 1


def kernel(x, w_mat):
    m_g, k_l = x.shape
    k_l2, n = w_mat.shape
    assert k_l == k_l2
    m_l = m_g // N_DEV
    n_half = n // 2
    n_rounds = n_half // RING_COLS

    x = x.astype(jnp.bfloat16)
    w = w_mat.astype(jnp.bfloat16)

    def body(x_ref, w_ref, out_ref, slots_r, slots_l,
             send_r, recv_r, send_l, recv_l,
             ax_slots, ax_send, ax_recv, copy_sem):
        my = lax.axis_index("i")
        left = lax.rem(my + N_DEV - 1, N_DEV)
        right = lax.rem(my + 1, N_DEV)

        barrier = pltpu.get_barrier_semaphore()

        def neighbor_barrier():
            for nbr in (left, right):
                pl.semaphore_signal(
                    barrier, inc=1, device_id=(nbr,),
                    device_id_type=pl.DeviceIdType.MESH,
                )
            pl.semaphore_wait(barrier, 2)

        def partial(b, c0):
            xs = x_ref[pl.ds(b * m_l, m_l), :]
            wt = w_ref[:, c0: c0 + RING_COLS]
            return jnp.dot(xs, wt, preferred_element_type=jnp.float32)

        neighbor_barrier()

        amax = jnp.float32(0.0)

        for k in range(n_rounds):
            c0r = k * RING_COLS
            c0l = n_half + k * RING_COLS
            if k > 0:
                neighbor_barrier()

            br0 = lax.rem(my + N_DEV - 1, N_DEV)
            bl0 = lax.rem(my + 1, N_DEV)
            slots_r[0, :, :] = partial(br0, c0r).astype(jnp.bfloat16)
            slots_l[0, :, :] = partial(bl0, c0l).astype(jnp.bfloat16)

            for h in range(N_HOPS):
                rr = pltpu.make_async_remote_copy(
                    src_ref=slots_r.at[h], dst_ref=slots_r.at[h + 1],
                    send_sem=send_r.at[h], recv_sem=recv_r.at[h],
                    device_id=(right,), device_id_type=pl.DeviceIdType.MESH,
                )
                ll = pltpu.make_async_remote_copy(
                    src_ref=slots_l.at[h], dst_ref=slots_l.at[h + 1],
                    send_sem=send_l.at[h], recv_sem=recv_l.at[h],
                    device_id=(left,), device_id_type=pl.DeviceIdType.MESH,
                )
                rr.start()
                ll.start()
                rr.wait()
                ll.wait()
                if h < N_HOPS - 1:
                    br = lax.rem(my + 2 * N_DEV - 2 - h, N_DEV)
                    bl = lax.rem(my + 2 + h, N_DEV)
                    accr = slots_r[h + 1, :, :].astype(jnp.float32) + partial(br, c0r)
                    slots_r[h + 1, :, :] = accr.astype(jnp.bfloat16)
                    accl = slots_l[h + 1, :, :].astype(jnp.float32) + partial(bl, c0l)
                    slots_l[h + 1, :, :] = accl.astype(jnp.bfloat16)

            yr = slots_r[N_HOPS, :, :].astype(jnp.float32) + partial(my, c0r)
            amax = jnp.maximum(amax, jnp.max(jnp.abs(yr)))
            slots_r[N_HOPS, :, :] = yr.astype(jnp.bfloat16)
            yl = slots_l[N_HOPS, :, :].astype(jnp.float32) + partial(my, c0l)
            amax = jnp.maximum(amax, jnp.max(jnp.abs(yl)))
            slots_l[N_HOPS, :, :] = yl.astype(jnp.bfloat16)

            cpr = pltpu.make_async_copy(
                slots_r.at[N_HOPS], out_ref.at[:, pl.ds(c0r, RING_COLS)], copy_sem
            )
            cpr.start()
            cpr.wait()
            cpl = pltpu.make_async_copy(
                slots_l.at[N_HOPS], out_ref.at[:, pl.ds(c0l, RING_COLS)], copy_sem
            )
            cpl.start()
            cpl.wait()

        own = jnp.full((8, 128), amax, jnp.float32)
        ax_slots[0, :, :] = own
        for h in range(N_HOPS):
            rdma = pltpu.make_async_remote_copy(
                src_ref=ax_slots.at[h], dst_ref=ax_slots.at[h + 1],
                send_sem=ax_send.at[h], recv_sem=ax_recv.at[h],
                device_id=(right,), device_id_type=pl.DeviceIdType.MESH,
            )
            rdma.start()
            rdma.wait()
            ax_slots[h + 1, :, :] = jnp.maximum(ax_slots[h + 1, :, :], own)
        g = jnp.max(ax_slots[N_HOPS, :, :])

        scale = g / 448.0
        inv = 448.0 / g
        for t in range(n // RING_COLS):
            c0 = t * RING_COLS
            cp = pltpu.make_async_copy(
                out_ref.at[:, pl.ds(c0, RING_COLS)], slots_r.at[0], copy_sem
            )
            cp.start()
            cp.wait()
            v = slots_r[0, :, :].astype(jnp.float32) * inv
            v = jnp.clip(v, -448.0, 448.0)
            q = v.astype(jnp.float8_e4m3fn).astype(jnp.float32)
            slots_r[0, :, :] = (q * scale).astype(jnp.bfloat16)
            cp = pltpu.make_async_copy(
                slots_r.at[0], out_ref.at[:, pl.ds(c0, RING_COLS)], copy_sem
            )
            cp.start()
            cp.wait()

    return pl.pallas_call(
        body,
        out_shape=jax.ShapeDtypeStruct((m_l, n), jnp.bfloat16),
        in_specs=[
            pl.BlockSpec(memory_space=pltpu.VMEM),
            pl.BlockSpec(memory_space=pltpu.VMEM),
        ],
        out_specs=pl.BlockSpec(memory_space=pl.ANY),
        scratch_shapes=[
            pltpu.VMEM((N_HOPS + 1, m_l, RING_COLS), jnp.bfloat16),
            pltpu.VMEM((N_HOPS + 1, m_l, RING_COLS), jnp.bfloat16),
            pltpu.SemaphoreType.DMA((N_HOPS,)),
            pltpu.SemaphoreType.DMA((N_HOPS,)),
            pltpu.SemaphoreType.DMA((N_HOPS,)),
            pltpu.SemaphoreType.DMA((N_HOPS,)),
            pltpu.VMEM((N_HOPS + 1, 8, 128), jnp.float32),
            pltpu.SemaphoreType.DMA((N_HOPS,)),
            pltpu.SemaphoreType.DMA((N_HOPS,)),
            pltpu.SemaphoreType.DMA,
        ],
        compiler_params=pltpu.CompilerParams(
            collective_id=0, vmem_limit_bytes=64 * 1024 * 1024
        ),
    )(x, w)


# device time: 368153 ns/iter; 1.9782x vs baseline; 1.2675x over previous
import jax
import jax.numpy as jnp
from jax import lax
from jax.experimental import pallas as pl
from jax.experimental.pallas import tpu as pltpu

N_DEV = 4
RING_COLS = 1024
SUB = RING_COLS // 2
N_HOPS = N_DEV - 1


def kernel(x, w_mat):
    m_g, k_l = x.shape
    k_l2, n = w_mat.shape
    assert k_l == k_l2
    m_l = m_g // N_DEV
    n_half = n // 2
    n_rounds = n_half // RING_COLS

    x = x.astype(jnp.bfloat16)
    w = w_mat.astype(jnp.bfloat16)

    def body(x_ref, w_ref, out_ref, slots_r, slots_l, y_vmem,
             send_r, recv_r, send_l, recv_l,
             ax_slots, ax_send, ax_recv, out_sems):
        my = lax.axis_index("i")
        left = lax.rem(my + N_DEV - 1, N_DEV)
        right = lax.rem(my + 1, N_DEV)

        barrier = pltpu.get_barrier_semaphore()

        def barrier_signal():
            for nbr in (left, right):
                pl.semaphore_signal(
                    barrier, inc=1, device_id=(nbr,),
                    device_id_type=pl.DeviceIdType.MESH,
                )

        def partial_bf16(b, c0):
            xs = x_ref[pl.ds(b * m_l, m_l), :]
            wt = w_ref[:, c0: c0 + RING_COLS]
            return jnp.dot(
                xs, wt, preferred_element_type=jnp.float32
            ).astype(jnp.bfloat16)

        def mk(slots, sems_s, sems_r, h, s, dev):
            return pltpu.make_async_remote_copy(
                src_ref=slots.at[h, s],
                dst_ref=slots.at[h + 1, s],
                send_sem=sems_s.at[h, s], recv_sem=sems_r.at[h, s],
                device_id=(dev,), device_id_type=pl.DeviceIdType.MESH,
            )

        def br(h):
            return lax.rem(my + 2 * N_DEV - 2 - h, N_DEV)

        def bl(h):
            return lax.rem(my + 2 + h, N_DEV)

        def store_lanes(slots, h, val):
            for s in range(2):
                slots[h, s, :, :] = val[:, s * SUB:(s + 1) * SUB]

        store_lanes(slots_r, 0, partial_bf16(lax.rem(my + N_DEV - 1, N_DEV), 0))
        store_lanes(slots_l, 0, partial_bf16(lax.rem(my + 1, N_DEV), n_half))

        barrier_signal()
        pl.semaphore_wait(barrier, 2)

        amax = jnp.float32(0.0)

        for k in range(n_rounds):
            c0r = k * RING_COLS
            c0l = n_half + k * RING_COLS
            if k > 0:
                pl.semaphore_wait(barrier, 2)

            rdma_r = {}
            rdma_l = {}
            for s in range(2):
                rdma_r[0, s] = mk(slots_r, send_r, recv_r, 0, s, right)
                rdma_l[0, s] = mk(slots_l, send_l, recv_l, 0, s, left)
                rdma_r[0, s].start()
                rdma_l[0, s].start()

            p_r = partial_bf16(br(0), c0r)
            p_l = partial_bf16(bl(0), c0l)

            for h in range(N_HOPS):
                for s in range(2):
                    lane = slice(s * SUB, (s + 1) * SUB)
                    rdma_r[h, s].wait()
                    if h < N_HOPS - 1:
                        slots_r[h + 1, s, :, :] = (
                            slots_r[h + 1, s, :, :] + p_r[:, lane]
                        )
                        rdma_r[h + 1, s] = mk(slots_r, send_r, recv_r, h + 1, s, right)
                        rdma_r[h + 1, s].start()
                    rdma_l[h, s].wait()
                    if h < N_HOPS - 1:
                        slots_l[h + 1, s, :, :] = (
                            slots_l[h + 1, s, :, :] + p_l[:, lane]
                        )
                        rdma_l[h + 1, s] = mk(slots_l, send_l, recv_l, h + 1, s, left)
                        rdma_l[h + 1, s].start()
                if h < N_HOPS - 1:
                    p_r = partial_bf16(br(h + 1), c0r)
                    p_l = partial_bf16(bl(h + 1), c0l)

            if k + 1 < n_rounds:
                barrier_signal()

            for s in range(2):
                lane = slice(s * SUB, (s + 1) * SUB)
                yr = slots_r[N_HOPS, s, :, :].astype(jnp.float32) + p_r[:, lane].astype(jnp.float32)
                amax = jnp.maximum(amax, jnp.max(jnp.abs(yr)))
                y_vmem[:, c0r + s * SUB: c0r + (s + 1) * SUB] = yr.astype(jnp.bfloat16)
                yl = slots_l[N_HOPS, s, :, :].astype(jnp.float32) + p_l[:, lane].astype(jnp.float32)
                amax = jnp.maximum(amax, jnp.max(jnp.abs(yl)))
                y_vmem[:, c0l + s * SUB: c0l + (s + 1) * SUB] = yl.astype(jnp.bfloat16)

            if k + 1 < n_rounds:
                store_lanes(slots_r, 0, partial_bf16(
                    lax.rem(my + N_DEV - 1, N_DEV), c0r + RING_COLS))
                store_lanes(slots_l, 0, partial_bf16(
                    lax.rem(my + 1, N_DEV), c0l + RING_COLS))

        own = jnp.full((8, 128), amax, jnp.float32)
        ax_slots[0, :, :] = own
        for h in range(N_HOPS):
            rdma = pltpu.make_async_remote_copy(
                src_ref=ax_slots.at[h], dst_ref=ax_slots.at[h + 1],
                send_sem=ax_send.at[h], recv_sem=ax_recv.at[h],
                device_id=(right,), device_id_type=pl.DeviceIdType.MESH,
            )
            rdma.start()
            rdma.wait()
            ax_slots[h + 1, :, :] = jnp.maximum(ax_slots[h + 1, :, :], own)
        g = jnp.max(ax_slots[N_HOPS, :, :])

        scale = g / 448.0
        inv = 448.0 / g
        cps = [None, None]
        for t in range(n // RING_COLS):
            b = t % 2
            if cps[b] is not None:
                for cp in cps[b]:
                    cp.wait()
            v = y_vmem[:, t * RING_COLS:(t + 1) * RING_COLS].astype(jnp.float32)
            v = jnp.clip(v * inv, -448.0, 448.0)
            q = v.astype(jnp.float8_e4m3fn).astype(jnp.float32)
            res = (q * scale).astype(jnp.bfloat16)
            store_lanes(slots_r, b, res)
            pair = []
            for s in range(2):
                cp = pltpu.make_async_copy(
                    slots_r.at[b, s],
                    out_ref.at[:, pl.ds(t * RING_COLS + s * SUB, SUB)],
                    out_sems.at[b, s],
                )
                cp.start()
                pair.append(cp)
            cps[b] = pair
        for pair in cps:
            for cp in pair:
                cp.wait()

    return pl.pallas_call(
        body,
        out_shape=jax.ShapeDtypeStruct((m_l, n), jnp.bfloat16),
        in_specs=[
            pl.BlockSpec(memory_space=pltpu.VMEM),
            pl.BlockSpec(memory_space=pltpu.VMEM),
        ],
        out_specs=pl.BlockSpec(memory_space=pl.ANY),
        scratch_shapes=[
            pltpu.VMEM((N_HOPS + 1, 2, m_l, SUB), jnp.bfloat16),
            pltpu.VMEM((N_HOPS + 1, 2, m_l, SUB), jnp.bfloat16),
            pltpu.VMEM((m_l, n), jnp.bfloat16),
            pltpu.SemaphoreType.DMA((N_HOPS, 2)),
            pltpu.SemaphoreType.DMA((N_HOPS, 2)),
            pltpu.SemaphoreType.DMA((N_HOPS, 2)),
            pltpu.SemaphoreType.DMA((N_HOPS, 2)),
            pltpu.VMEM((N_HOPS + 1, 8, 128), jnp.float32),
            pltpu.SemaphoreType.DMA((N_HOPS,)),
            pltpu.SemaphoreType.DMA((N_HOPS,)),
            pltpu.SemaphoreType.DMA((2, 2)),
        ],
        compiler_params=pltpu.CompilerParams(
            collective_id=0, vmem_limit_bytes=64 * 1024 * 1024
        ),
    )(x, w)


# device time: 364912 ns/iter; 1.9957x vs baseline; 1.0089x over previous
import jax
import jax.numpy as jnp
from jax import lax
from jax.experimental import pallas as pl
from jax.experimental.pallas import tpu as pltpu

N_DEV = 4
RING_COLS = 1024
SUB = RING_COLS // 2
N_HOPS = N_DEV - 1


def kernel(x, w_mat):
    m_g, k_l = x.shape
    k_l2, n = w_mat.shape
    assert k_l == k_l2
    m_l = m_g // N_DEV
    n_half = n // 2
    n_rounds = n_half // RING_COLS

    x = x.astype(jnp.bfloat16)
    w = w_mat.astype(jnp.bfloat16)

    def body(x_ref, w_ref, out_ref, slots_r, slots_l, y_vmem,
             send_r, recv_r, send_l, recv_l,
             ax_slots, ax_send, ax_recv, out_sems, cred_r, cred_l):
        my = lax.axis_index("i")
        left = lax.rem(my + N_DEV - 1, N_DEV)
        right = lax.rem(my + 1, N_DEV)

        barrier = pltpu.get_barrier_semaphore()

        def credit(sem, nbr):
            pl.semaphore_signal(
                sem, inc=1, device_id=(nbr,),
                device_id_type=pl.DeviceIdType.MESH,
            )

        def barrier_signal():
            for nbr in (left, right):
                pl.semaphore_signal(
                    barrier, inc=1, device_id=(nbr,),
                    device_id_type=pl.DeviceIdType.MESH,
                )

        def partial_bf16(b, c0):
            xs = x_ref[pl.ds(b * m_l, m_l), :]
            wt = w_ref[:, c0: c0 + RING_COLS]
            return jnp.dot(
                xs, wt, preferred_element_type=jnp.float32
            ).astype(jnp.bfloat16)

        def mk(slots, sems_s, sems_r, h, s, dev):
            return pltpu.make_async_remote_copy(
                src_ref=slots.at[h, s],
                dst_ref=slots.at[h + 1, s],
                send_sem=sems_s.at[h, s], recv_sem=sems_r.at[h, s],
                device_id=(dev,), device_id_type=pl.DeviceIdType.MESH,
            )

        def br(h):
            return lax.rem(my + 2 * N_DEV - 2 - h, N_DEV)

        def bl(h):
            return lax.rem(my + 2 + h, N_DEV)

        def store_lanes(slots, h, val):
            for s in range(2):
                slots[h, s, :, :] = val[:, s * SUB:(s + 1) * SUB]

        store_lanes(slots_r, 0, partial_bf16(lax.rem(my + N_DEV - 1, N_DEV), 0))
        store_lanes(slots_l, 0, partial_bf16(lax.rem(my + 1, N_DEV), n_half))

        barrier_signal()
        pl.semaphore_wait(barrier, 2)

        amax = jnp.float32(0.0)

        rdma_r = {}
        rdma_l = {}
        for s in range(2):
            rdma_r[0, s] = mk(slots_r, send_r, recv_r, 0, s, right)
            rdma_l[0, s] = mk(slots_l, send_l, recv_l, 0, s, left)
            rdma_r[0, s].start()
            rdma_l[0, s].start()

        p_r = partial_bf16(br(0), 0)
        p_l = partial_bf16(bl(0), n_half)

        for k in range(n_rounds):
            c0r = k * RING_COLS
            c0l = n_half + k * RING_COLS
            more = k + 1 < n_rounds

            for h in range(N_HOPS):
                for s in range(2):
                    rdma_r[h, s].wait()
                    if h >= 1 and more:
                        credit(cred_r, left)
                    if h < N_HOPS - 1:
                        slots_r[h + 1, s, :, :] = (
                            slots_r[h + 1, s, :, :] + p_r[:, s * SUB:(s + 1) * SUB]
                        )
                        if k > 0:
                            pl.semaphore_wait(cred_r, 1)
                        rdma_r[h + 1, s] = mk(slots_r, send_r, recv_r, h + 1, s, right)
                        rdma_r[h + 1, s].start()
                    rdma_l[h, s].wait()
                    if h >= 1 and more:
                        credit(cred_l, right)
                    if h < N_HOPS - 1:
                        slots_l[h + 1, s, :, :] = (
                            slots_l[h + 1, s, :, :] + p_l[:, s * SUB:(s + 1) * SUB]
                        )
                        if k > 0:
                            pl.semaphore_wait(cred_l, 1)
                        rdma_l[h + 1, s] = mk(slots_l, send_l, recv_l, h + 1, s, left)
                        rdma_l[h + 1, s].start()
                if h < N_HOPS - 1:
                    p_r = partial_bf16(br(h + 1), c0r)
                    p_l = partial_bf16(bl(h + 1), c0l)

            fin_pr, fin_pl = p_r, p_l
            if more:
                store_lanes(slots_r, 0, partial_bf16(
                    lax.rem(my + N_DEV - 1, N_DEV), c0r + RING_COLS))
                store_lanes(slots_l, 0, partial_bf16(
                    lax.rem(my + 1, N_DEV), c0l + RING_COLS))
                for s in range(2):
                    pl.semaphore_wait(cred_r, 1)
                    rdma_r[0, s] = mk(slots_r, send_r, recv_r, 0, s, right)
                    rdma_r[0, s].start()
                    pl.semaphore_wait(cred_l, 1)
                    rdma_l[0, s] = mk(slots_l, send_l, recv_l, 0, s, left)
                    rdma_l[0, s].start()
                p_r = partial_bf16(br(0), c0r + RING_COLS)
                p_l = partial_bf16(bl(0), c0l + RING_COLS)

            for s in range(2):
                lane = slice(s * SUB, (s + 1) * SUB)
                yr = slots_r[N_HOPS, s, :, :].astype(jnp.float32) + fin_pr[:, lane].astype(jnp.float32)
                amax = jnp.maximum(amax, jnp.max(jnp.abs(yr)))
                y_vmem[:, c0r + s * SUB: c0r + (s + 1) * SUB] = yr.astype(jnp.bfloat16)
                if more:
                    credit(cred_r, left)
                yl = slots_l[N_HOPS, s, :, :].astype(jnp.float32) + fin_pl[:, lane].astype(jnp.float32)
                amax = jnp.maximum(amax, jnp.max(jnp.abs(yl)))
                y_vmem[:, c0l + s * SUB: c0l + (s + 1) * SUB] = yl.astype(jnp.bfloat16)
                if more:
                    credit(cred_l, right)

        own = jnp.full((8, 128), amax, jnp.float32)
        ax_slots[0, :, :] = own
        for h in range(N_HOPS):
            rdma = pltpu.make_async_remote_copy(
                src_ref=ax_slots.at[h], dst_ref=ax_slots.at[h + 1],
                send_sem=ax_send.at[h], recv_sem=ax_recv.at[h],
                device_id=(right,), device_id_type=pl.DeviceIdType.MESH,
            )
            rdma.start()
            rdma.wait()
            ax_slots[h + 1, :, :] = jnp.maximum(ax_slots[h + 1, :, :], own)
        g = jnp.max(ax_slots[N_HOPS, :, :])

        scale = g / 448.0
        inv = 448.0 / g
        cps = [None, None]
        for t in range(n // RING_COLS):
            b = t % 2
            if cps[b] is not None:
                for cp in cps[b]:
                    cp.wait()
            v = y_vmem[:, t * RING_COLS:(t + 1) * RING_COLS].astype(jnp.float32)
            v = jnp.clip(v * inv, -448.0, 448.0)
            q = v.astype(jnp.float8_e4m3fn).astype(jnp.float32)
            res = (q * scale).astype(jnp.bfloat16)
            store_lanes(slots_r, b, res)
            pair = []
            for s in range(2):
                cp = pltpu.make_async_copy(
                    slots_r.at[b, s],
                    out_ref.at[:, pl.ds(t * RING_COLS + s * SUB, SUB)],
                    out_sems.at[b, s],
                )
                cp.start()
                pair.append(cp)
            cps[b] = pair
        for pair in cps:
            for cp in pair:
                cp.wait()

    return pl.pallas_call(
        body,
        out_shape=jax.ShapeDtypeStruct((m_l, n), jnp.bfloat16),
        in_specs=[
            pl.BlockSpec(memory_space=pltpu.VMEM),
            pl.BlockSpec(memory_space=pltpu.VMEM),
        ],
        out_specs=pl.BlockSpec(memory_space=pl.ANY),
        scratch_shapes=[
            pltpu.VMEM((N_HOPS + 1, 2, m_l, SUB), jnp.bfloat16),
            pltpu.VMEM((N_HOPS + 1, 2, m_l, SUB), jnp.bfloat16),
            pltpu.VMEM((m_l, n), jnp.bfloat16),
            pltpu.SemaphoreType.DMA((N_HOPS, 2)),
            pltpu.SemaphoreType.DMA((N_HOPS, 2)),
            pltpu.SemaphoreType.DMA((N_HOPS, 2)),
            pltpu.SemaphoreType.DMA((N_HOPS, 2)),
            pltpu.VMEM((N_HOPS + 1, 8, 128), jnp.float32),
            pltpu.SemaphoreType.DMA((N_HOPS,)),
            pltpu.SemaphoreType.DMA((N_HOPS,)),
            pltpu.SemaphoreType.DMA((2, 2)),
            pltpu.SemaphoreType.REGULAR,
            pltpu.SemaphoreType.REGULAR,
        ],
        compiler_params=pltpu.CompilerParams(
            collective_id=0, vmem_limit_bytes=64 * 1024 * 1024
        ),
    )(x, w)


# device time: 363146 ns/iter; 2.0054x vs baseline; 1.0049x over previous
import jax
import jax.numpy as jnp
from jax import lax
from jax.experimental import pallas as pl
from jax.experimental.pallas import tpu as pltpu

N_DEV = 4
RING_COLS = 1024
N_LANES = 2
SUB = RING_COLS // N_LANES
N_HOPS = N_DEV - 1


def kernel(x, w_mat):
    m_g, k_l = x.shape
    k_l2, n = w_mat.shape
    assert k_l == k_l2
    m_l = m_g // N_DEV
    n_half = n // 2
    n_rounds = n_half // RING_COLS

    x = x.astype(jnp.bfloat16)
    w = w_mat.astype(jnp.bfloat16)

    def body(x_ref, w_ref, out_ref, slots_r, slots_l, y_vmem,
             send_r, recv_r, send_l, recv_l,
             ax_slots, ax_send, ax_recv, out_sems, cred_r, cred_l):
        my = lax.axis_index("i")
        left = lax.rem(my + N_DEV - 1, N_DEV)
        right = lax.rem(my + 1, N_DEV)

        barrier = pltpu.get_barrier_semaphore()

        def credit(sem, nbr):
            pl.semaphore_signal(
                sem, inc=1, device_id=(nbr,),
                device_id_type=pl.DeviceIdType.MESH,
            )

        def barrier_signal():
            for nbr in (left, right):
                pl.semaphore_signal(
                    barrier, inc=1, device_id=(nbr,),
                    device_id_type=pl.DeviceIdType.MESH,
                )

        def partial_bf16(b, c0):
            xs = x_ref[pl.ds(b * m_l, m_l), :]
            wt = w_ref[:, c0: c0 + RING_COLS]
            return jnp.dot(
                xs, wt, preferred_element_type=jnp.float32
            ).astype(jnp.bfloat16)

        def mk(slots, sems_s, sems_r, h, s, dev):
            return pltpu.make_async_remote_copy(
                src_ref=slots.at[h, s],
                dst_ref=slots.at[h + 1, s],
                send_sem=sems_s.at[h, s], recv_sem=sems_r.at[h, s],
                device_id=(dev,), device_id_type=pl.DeviceIdType.MESH,
            )

        def br(h):
            return lax.rem(my + 2 * N_DEV - 2 - h, N_DEV)

        def bl(h):
            return lax.rem(my + 2 + h, N_DEV)

        def store_lanes(slots, h, val):
            for s in range(N_LANES):
                slots[h, s, :, :] = val[:, s * SUB:(s + 1) * SUB]

        store_lanes(slots_r, 0, partial_bf16(lax.rem(my + N_DEV - 1, N_DEV), 0))
        store_lanes(slots_l, 0, partial_bf16(lax.rem(my + 1, N_DEV), n_half))

        barrier_signal()
        pl.semaphore_wait(barrier, 2)

        amax = jnp.float32(0.0)

        rdma_r = {}
        rdma_l = {}
        for s in range(N_LANES):
            rdma_r[0, s] = mk(slots_r, send_r, recv_r, 0, s, right)
            rdma_l[0, s] = mk(slots_l, send_l, recv_l, 0, s, left)
            rdma_r[0, s].start()
            rdma_l[0, s].start()

        p_r = partial_bf16(br(0), 0)
        p_l = partial_bf16(bl(0), n_half)

        for k in range(n_rounds):
            c0r = k * RING_COLS
            c0l = n_half + k * RING_COLS
            more = k + 1 < n_rounds

            for h in range(N_HOPS):
                for s in range(N_LANES):
                    rdma_r[h, s].wait()
                    if h >= 1 and more:
                        credit(cred_r, left)
                    if h < N_HOPS - 1:
                        slots_r[h + 1, s, :, :] = (
                            slots_r[h + 1, s, :, :] + p_r[:, s * SUB:(s + 1) * SUB]
                        )
                        if k > 0:
                            pl.semaphore_wait(cred_r, 1)
                        rdma_r[h + 1, s] = mk(slots_r, send_r, recv_r, h + 1, s, right)
                        rdma_r[h + 1, s].start()
                    rdma_l[h, s].wait()
                    if h >= 1 and more:
                        credit(cred_l, right)
                    if h < N_HOPS - 1:
                        slots_l[h + 1, s, :, :] = (
                            slots_l[h + 1, s, :, :] + p_l[:, s * SUB:(s + 1) * SUB]
                        )
                        if k > 0:
                            pl.semaphore_wait(cred_l, 1)
                        rdma_l[h + 1, s] = mk(slots_l, send_l, recv_l, h + 1, s, left)
                        rdma_l[h + 1, s].start()
                if h < N_HOPS - 1:
                    p_r = partial_bf16(br(h + 1), c0r)
                    p_l = partial_bf16(bl(h + 1), c0l)

            fin_pr, fin_pl = p_r, p_l
            if more:
                store_lanes(slots_r, 0, partial_bf16(
                    lax.rem(my + N_DEV - 1, N_DEV), c0r + RING_COLS))
                store_lanes(slots_l, 0, partial_bf16(
                    lax.rem(my + 1, N_DEV), c0l + RING_COLS))
                for s in range(N_LANES):
                    pl.semaphore_wait(cred_r, 1)
                    rdma_r[0, s] = mk(slots_r, send_r, recv_r, 0, s, right)
                    rdma_r[0, s].start()
                    pl.semaphore_wait(cred_l, 1)
                    rdma_l[0, s] = mk(slots_l, send_l, recv_l, 0, s, left)
                    rdma_l[0, s].start()
                p_r = partial_bf16(br(0), c0r + RING_COLS)
                p_l = partial_bf16(bl(0), c0l + RING_COLS)

            for s in range(N_LANES):
                lane = slice(s * SUB, (s + 1) * SUB)
                yr = slots_r[N_HOPS, s, :, :].astype(jnp.float32) + fin_pr[:, lane].astype(jnp.float32)
                amax = jnp.maximum(amax, jnp.max(jnp.abs(yr)))
                y_vmem[:, c0r + s * SUB: c0r + (s + 1) * SUB] = yr.astype(jnp.bfloat16)
                if more:
                    credit(cred_r, left)
                yl = slots_l[N_HOPS, s, :, :].astype(jnp.float32) + fin_pl[:, lane].astype(jnp.float32)
                amax = jnp.maximum(amax, jnp.max(jnp.abs(yl)))
                y_vmem[:, c0l + s * SUB: c0l + (s + 1) * SUB] = yl.astype(jnp.bfloat16)
                if more:
                    credit(cred_l, right)

        own = jnp.full((8, 128), amax, jnp.float32)
        ax_slots[0, :, :] = own
        a1 = pltpu.make_async_remote_copy(
            src_ref=ax_slots.at[0], dst_ref=ax_slots.at[1],
            send_sem=ax_send.at[0], recv_sem=ax_recv.at[0],
            device_id=(right,), device_id_type=pl.DeviceIdType.MESH,
        )
        a2 = pltpu.make_async_remote_copy(
            src_ref=ax_slots.at[0], dst_ref=ax_slots.at[2],
            send_sem=ax_send.at[1], recv_sem=ax_recv.at[1],
            device_id=(left,), device_id_type=pl.DeviceIdType.MESH,
        )
        a1.start()
        a2.start()
        a1.wait()
        a2.wait()
        ax_slots[3, :, :] = jnp.maximum(ax_slots[1, :, :], own)
        b1 = pltpu.make_async_remote_copy(
            src_ref=ax_slots.at[3], dst_ref=ax_slots.at[4],
            send_sem=ax_send.at[2], recv_sem=ax_recv.at[2],
            device_id=(right,), device_id_type=pl.DeviceIdType.MESH,
        )
        b1.start()
        b1.wait()
        g = jnp.max(
            jnp.maximum(ax_slots[4, :, :],
                        jnp.maximum(ax_slots[2, :, :], own))
        )

        scale = g / 448.0
        inv = 448.0 / g
        cps = [None, None]
        for t in range(n // RING_COLS):
            b = t % 2
            if cps[b] is not None:
                for cp in cps[b]:
                    cp.wait()
            v = y_vmem[:, t * RING_COLS:(t + 1) * RING_COLS].astype(jnp.float32)
            v = jnp.clip(v * inv, -448.0, 448.0)
            q = v.astype(jnp.float8_e4m3fn).astype(jnp.float32)
            res = (q * scale).astype(jnp.bfloat16)
            store_lanes(slots_r, b, res)
            pair = []
            for s in range(N_LANES):
                cp = pltpu.make_async_copy(
                    slots_r.at[b, s],
                    out_ref.at[:, pl.ds(t * RING_COLS + s * SUB, SUB)],
                    out_sems.at[b, s],
                )
                cp.start()
                pair.append(cp)
            cps[b] = pair
        for pair in cps:
            for cp in pair:
                cp.wait()

    return pl.pallas_call(
        body,
        out_shape=jax.ShapeDtypeStruct((m_l, n), jnp.bfloat16),
        in_specs=[
            pl.BlockSpec(memory_space=pltpu.VMEM),
            pl.BlockSpec(memory_space=pltpu.VMEM),
        ],
        out_specs=pl.BlockSpec(memory_space=pl.ANY),
        scratch_shapes=[
            pltpu.VMEM((N_HOPS + 1, N_LANES, m_l, SUB), jnp.bfloat16),
            pltpu.VMEM((N_HOPS + 1, N_LANES, m_l, SUB), jnp.bfloat16),
            pltpu.VMEM((m_l, n), jnp.bfloat16),
            pltpu.SemaphoreType.DMA((N_HOPS, N_LANES)),
            pltpu.SemaphoreType.DMA((N_HOPS, N_LANES)),
            pltpu.SemaphoreType.DMA((N_HOPS, N_LANES)),
            pltpu.SemaphoreType.DMA((N_HOPS, N_LANES)),
            pltpu.VMEM((5, 8, 128), jnp.float32),
            pltpu.SemaphoreType.DMA((N_HOPS,)),
            pltpu.SemaphoreType.DMA((N_HOPS,)),
            pltpu.SemaphoreType.DMA((2, N_LANES)),
            pltpu.SemaphoreType.REGULAR,
            pltpu.SemaphoreType.REGULAR,
        ],
        compiler_params=pltpu.CompilerParams(
            collective_id=0, vmem_limit_bytes=64 * 1024 * 1024
        ),
    )(x, w)
